# baseline (speedup 1.0000x reference)
import jax
import jax.numpy as jnp
from jax.experimental import pallas as pl


def _tanh2_block(x_ref, o_ref):
    o_ref[...] = jnp.tanh(jnp.tanh(x_ref[...]))


def kernel(x, neuron_states, enc_W, enc_b, dec_W, dec_b):
    batch, num_neurons = x.shape
    block_rows = 64
    grid = (batch // block_rows,)
    return pl.pallas_call(
        _tanh2_block,
        grid=grid,
        in_specs=[pl.BlockSpec((block_rows, num_neurons), lambda i: (i, 0))],
        out_specs=pl.BlockSpec((block_rows, num_neurons), lambda i: (i, 0)),
        out_shape=jax.ShapeDtypeStruct((batch, num_neurons), x.dtype),
    )(x)


# speedup vs baseline: 1.3386x; 1.3386x over previous
import jax
import jax.numpy as jnp
from jax.experimental import pallas as pl


def _tanh2_block(x_ref, o_ref):
    o_ref[...] = jnp.tanh(jnp.tanh(x_ref[...]))


def kernel(x, neuron_states, enc_W, enc_b, dec_W, dec_b):
    batch, num_neurons = x.shape
    block_rows = 32
    grid = (batch // block_rows,)
    return pl.pallas_call(
        _tanh2_block,
        grid=grid,
        in_specs=[pl.BlockSpec((block_rows, num_neurons), lambda i: (i, 0))],
        out_specs=pl.BlockSpec((block_rows, num_neurons), lambda i: (i, 0)),
        out_shape=jax.ShapeDtypeStruct((batch, num_neurons), x.dtype),
    )(x)
